# baseline (speedup 1.0000x reference)
import functools

import jax
import jax.numpy as jnp
from jax import lax
from jax.experimental import pallas as pl
from jax.experimental.pallas import tpu as pltpu
from jax.experimental.pallas import tpu_sc as plsc

B = 16384
WIN = 48
WOUT = 63
OFFS = (0, 16, 32, 47)

B_SC = 6144
NWORK = 32
RPW = B_SC // NWORK

B_TC = B - B_SC
BLK = 2048


def _sc_body(cm_ref, x_hbm, o_hbm, in_v, out_v, cm_v):
    wid = lax.axis_index("s") * 2 + lax.axis_index("c")
    base = wid * RPW
    pltpu.sync_copy(cm_ref, cm_v)
    cols = [cm_v[pl.ds(k * 16, 16)] for k in range(4)]
    pltpu.sync_copy(x_hbm.at[pl.ds(base, RPW), :], in_v)

    def row(r, _):
        rv = jnp.full((16,), r, jnp.int32)
        for k in range(4):
            out_v[r, pl.ds(OFFS[k], 16)] = plsc.load_gather(
                in_v, [rv, cols[k]])
        return _

    lax.fori_loop(0, RPW, row, None)
    pltpu.sync_copy(out_v, o_hbm.at[pl.ds(base, RPW), :])


def _sc_call(in2d, cm):
    f = functools.partial(
        pl.kernel,
        out_type=jax.ShapeDtypeStruct((B_SC, WOUT), jnp.float32),
        mesh=plsc.VectorSubcoreMesh(core_axis_name="c", subcore_axis_name="s"),
        scratch_types=[
            pltpu.VMEM((RPW, WIN), jnp.float32),
            pltpu.VMEM((RPW, WOUT), jnp.float32),
            pltpu.VMEM((64,), jnp.int32),
        ],
        compiler_params=pltpu.CompilerParams(
            needs_layout_passes=False, use_tc_tiling_on_sc=True),
    )(_sc_body)
    return f(cm, in2d)


def _tc_body(cmap_ref, x_ref, o_ref):
    rows = lax.broadcasted_iota(jnp.int32, (WIN, WOUT), 0)
    g = (rows == cmap_ref[...]).astype(jnp.float32)
    o_ref[...] = lax.dot_general(
        x_ref[...], g, (((1,), (0,)), ((), ())),
        preferred_element_type=jnp.float32,
        precision=lax.Precision.DEFAULT)


def _tc_call(in2d, cmap):
    return pl.pallas_call(
        _tc_body,
        grid=(B_TC // BLK,),
        in_specs=[
            pl.BlockSpec((1, WOUT), lambda i: (0, 0)),
            pl.BlockSpec((BLK, WIN), lambda i: (i, 0)),
        ],
        out_specs=pl.BlockSpec((BLK, WOUT), lambda i: (i, 0)),
        out_shape=jax.ShapeDtypeStruct((B_TC, WOUT), jnp.float32),
        compiler_params=pltpu.CompilerParams(
            dimension_semantics=("parallel",)),
    )(cmap, in2d)


def kernel(joints, indices):
    cmap = (3 * jnp.repeat(indices.astype(jnp.int32), 3)
            + jnp.tile(jnp.arange(3, dtype=jnp.int32), 21))
    cm_sc = jnp.concatenate(
        [cmap[o:o + 16] for o in OFFS]).astype(jnp.int32)
    in2d = joints.reshape(B, WIN)
    out_sc = _sc_call(in2d[:B_SC], cm_sc)
    out_tc = _tc_call(in2d[B_SC:], cmap.reshape(1, WOUT))
    out2d = jnp.concatenate([out_sc, out_tc], axis=0)
    return out2d.reshape(B, 21, 3)


# speedup vs baseline: 1.0315x; 1.0315x over previous
import functools

import jax
import jax.numpy as jnp
from jax import lax
from jax.experimental import pallas as pl
from jax.experimental.pallas import tpu as pltpu
from jax.experimental.pallas import tpu_sc as plsc

B = 16384
WIN = 48
WOUT = 63
NWORK = 32
RPW = B // NWORK
CHR = 256
OFFS = (0, 16, 32, 47)


def _sc_body(cm_ref, x_hbm, o_hbm, in_v, out_v, cm_v):
    wid = lax.axis_index("s") * 2 + lax.axis_index("c")
    pltpu.sync_copy(cm_ref, cm_v)
    cols = [cm_v[pl.ds(k * 16, 16)] for k in range(4)]

    def row(r, _):
        rv = jnp.full((16,), r, jnp.int32)
        for k in range(4):
            out_v[r, pl.ds(OFFS[k], 16)] = plsc.load_gather(
                in_v, [rv, cols[k]])
        return _

    for c in range(RPW // CHR):
        base = wid * RPW + c * CHR
        pltpu.sync_copy(x_hbm.at[pl.ds(base, CHR), :], in_v)
        lax.fori_loop(0, CHR, row, None)
        pltpu.sync_copy(out_v, o_hbm.at[pl.ds(base, CHR), :])


def _sc_call(in2d, cm):
    f = functools.partial(
        pl.kernel,
        out_type=jax.ShapeDtypeStruct((B, WOUT), jnp.float32),
        mesh=plsc.VectorSubcoreMesh(core_axis_name="c", subcore_axis_name="s"),
        scratch_types=[
            pltpu.VMEM((CHR, WIN), jnp.float32),
            pltpu.VMEM((CHR, WOUT), jnp.float32),
            pltpu.VMEM((64,), jnp.int32),
        ],
        compiler_params=pltpu.CompilerParams(
            needs_layout_passes=False, use_tc_tiling_on_sc=True),
    )(_sc_body)
    return f(cm, in2d)


def kernel(joints, indices):
    cmap = (3 * jnp.repeat(indices.astype(jnp.int32), 3)
            + jnp.tile(jnp.arange(3, dtype=jnp.int32), 21))
    cm = jnp.concatenate(
        [cmap[o:o + 16] for o in OFFS]).astype(jnp.int32)
    out2d = _sc_call(joints.reshape(B, WIN), cm)
    return out2d.reshape(B, 21, 3)
